# baseline (speedup 1.0000x reference)
import functools
import math

import jax
import jax.numpy as jnp
from jax import lax
from jax.experimental import pallas as pl
from jax.experimental.pallas import tpu as pltpu
from jax.experimental.pallas import tpu_sc as plsc

D = 128
SCALE = math.sqrt(float(D))
NC = 2
NS = 16
NW = NC * NS
LANES = 16

NBUF = 2
LA = 1
CH_TOK = 400
GATHER_SPLITS = ((0, CH_TOK),)


@jax.jit
def _emb_lookup(tokens_flat, table):
    B = tokens_flat.shape[0]
    tok_per_w = B // NW
    n_ch = tok_per_w // CH_TOK

    mesh = plsc.VectorSubcoreMesh(core_axis_name="c", subcore_axis_name="s")

    @functools.partial(
        pl.kernel,
        out_type=jax.ShapeDtypeStruct((B, D), jnp.float32),
        mesh=mesh,
        compiler_params=pltpu.CompilerParams(use_tc_tiling_on_sc=True),
        scratch_types=[
            pltpu.VMEM((tok_per_w,), jnp.int32),
        ]
        + [pltpu.VMEM((CH_TOK, D), jnp.float32) for _ in range(NBUF)]
        + [pltpu.SemaphoreType.DMA for _ in range(2 * NBUF)],
    )
    def emb_kernel(tok_hbm, table_hbm, out_hbm, idx_v, *bufs_sems):
        bufs = bufs_sems[:NBUF]
        gsems = bufs_sems[NBUF:2 * NBUF]
        wsems = bufs_sems[2 * NBUF:]

        wid = lax.axis_index("s") * NC + lax.axis_index("c")
        base_tok = wid * tok_per_w
        pltpu.sync_copy(tok_hbm.at[pl.ds(base_tok, tok_per_w)], idx_v)

        def gather_descs(c, b):
            off = c * CH_TOK
            descs = [
                pltpu.make_async_copy(
                    table_hbm.at[idx_v.at[pl.ds(off + o, n)]],
                    bufs[b].at[pl.ds(o, n), :], gsems[b])
                for o, n in GATHER_SPLITS
            ]
            return descs

        def issue_gather(c, b):
            for d in gather_descs(c, b):
                d.start()

        def wait_gather(c, b):
            for d in gather_descs(c, b):
                d.wait()

        def scale(b):
            buf = bufs[b]

            def body(i, carry):
                for sub in range(D // LANES):
                    sl = pl.ds(sub * LANES, LANES)
                    buf[i, sl] = buf[i, sl] * SCALE
                return carry

            lax.fori_loop(0, CH_TOK, body, 0, unroll=2)

        def write_desc(c, b):
            return pltpu.make_async_copy(
                bufs[b],
                out_hbm.at[pl.ds(base_tok + c * CH_TOK, CH_TOK)], wsems[b])

        def step(c, b):
            wait_gather(c, b)
            scale(b)
            write_desc(c, b).start()

        for c in range(LA):
            issue_gather(c, c % NBUF)

        for c in range(NBUF):
            step(c, c)
            cn = c + LA
            if cn < NBUF:
                issue_gather(cn, cn)
            else:
                bn = cn % NBUF
                write_desc(cn - NBUF, bn).wait()
                issue_gather(cn, bn)

        def group(g, carry):
            c0 = g * NBUF
            for b in range(NBUF):
                c = c0 + b
                step(c, b)
                cn = c + LA
                bn = (b + LA) % NBUF

                @pl.when(cn < n_ch)
                def _():
                    write_desc(cn - NBUF, bn).wait()
                    issue_gather(cn, bn)
            return carry

        lax.fori_loop(1, n_ch // NBUF, group, 0)

        for b in range(NBUF):
            write_desc(0, b).wait()

    return emb_kernel(tokens_flat, table)


def kernel(tokens, table):
    rows, cols = tokens.shape
    tok_t = tokens.T.reshape(rows * cols).astype(jnp.int32)
    out = _emb_lookup(tok_t, table)
    return out.reshape(cols, rows, D).transpose(1, 0, 2)


# speedup vs baseline: 1.2431x; 1.2431x over previous
import functools
import math

import jax
import jax.numpy as jnp
from jax import lax
from jax.experimental import pallas as pl
from jax.experimental.pallas import tpu as pltpu
from jax.experimental.pallas import tpu_sc as plsc

D = 128
SCALE = math.sqrt(float(D))
NC = 2
NS = 16
NW = NC * NS
LANES = 16

NBUF = 5
LA = 3
CH_TOK = 128
GATHER_SPLITS = ((0, CH_TOK),)


@jax.jit
def _emb_lookup(tokens_flat, table):
    B = tokens_flat.shape[0]
    tok_per_w = B // NW
    n_ch = tok_per_w // CH_TOK

    mesh = plsc.VectorSubcoreMesh(core_axis_name="c", subcore_axis_name="s")

    @functools.partial(
        pl.kernel,
        out_type=jax.ShapeDtypeStruct((B, D), jnp.float32),
        mesh=mesh,
        compiler_params=pltpu.CompilerParams(use_tc_tiling_on_sc=True),
        scratch_types=[
            pltpu.VMEM((tok_per_w,), jnp.int32),
        ]
        + [pltpu.VMEM((CH_TOK, D), jnp.float32) for _ in range(NBUF)]
        + [pltpu.SemaphoreType.DMA for _ in range(2 * NBUF)],
    )
    def emb_kernel(tok_hbm, table_hbm, out_hbm, idx_v, *bufs_sems):
        bufs = bufs_sems[:NBUF]
        gsems = bufs_sems[NBUF:2 * NBUF]
        wsems = bufs_sems[2 * NBUF:]

        wid = lax.axis_index("s") * NC + lax.axis_index("c")
        base_tok = wid * tok_per_w
        pltpu.sync_copy(tok_hbm.at[pl.ds(base_tok, tok_per_w)], idx_v)

        def gather_descs(c, b):
            off = c * CH_TOK
            descs = [
                pltpu.make_async_copy(
                    table_hbm.at[idx_v.at[pl.ds(off + o, n)]],
                    bufs[b].at[pl.ds(o, n), :], gsems[b])
                for o, n in GATHER_SPLITS
            ]
            return descs

        def issue_gather(c, b):
            for d in gather_descs(c, b):
                d.start()

        def wait_gather(c, b):
            for d in gather_descs(c, b):
                d.wait()

        def scale(b):
            buf = bufs[b]

            def body(i, carry):
                for sub in range(D // LANES):
                    sl = pl.ds(sub * LANES, LANES)
                    buf[i, sl] = buf[i, sl] * SCALE
                return carry

            lax.fori_loop(0, CH_TOK, body, 0, unroll=2)

        def write_desc(c, b):
            return pltpu.make_async_copy(
                bufs[b],
                out_hbm.at[pl.ds(base_tok + c * CH_TOK, CH_TOK)], wsems[b])

        def step(c, b):
            wait_gather(c, b)
            scale(b)
            write_desc(c, b).start()

        for c in range(LA):
            issue_gather(c, c % NBUF)

        for c in range(NBUF):
            step(c, c)
            cn = c + LA
            if cn < NBUF:
                issue_gather(cn, cn)
            else:
                bn = cn % NBUF
                write_desc(cn - NBUF, bn).wait()
                issue_gather(cn, bn)

        def group(g, carry):
            c0 = g * NBUF
            for b in range(NBUF):
                c = c0 + b
                step(c, b)
                cn = c + LA
                bn = (b + LA) % NBUF

                @pl.when(cn < n_ch)
                def _():
                    write_desc(cn - NBUF, bn).wait()
                    issue_gather(cn, bn)
            return carry

        lax.fori_loop(1, n_ch // NBUF, group, 0)

        for b in range(NBUF):
            write_desc(0, b).wait()

    return emb_kernel(tokens_flat, table)


def kernel(tokens, table):
    rows, cols = tokens.shape
    tok_t = tokens.T.reshape(rows * cols).astype(jnp.int32)
    out = _emb_lookup(tok_t, table)
    return out.reshape(cols, rows, D).transpose(1, 0, 2)


# speedup vs baseline: 1.2432x; 1.0002x over previous
import functools
import math

import jax
import jax.numpy as jnp
from jax import lax
from jax.experimental import pallas as pl
from jax.experimental.pallas import tpu as pltpu
from jax.experimental.pallas import tpu_sc as plsc

D = 128
SCALE = math.sqrt(float(D))
NC = 2
NS = 16
NW = NC * NS
LANES = 16

NBUF = 5
LA = 4
CH_TOK = 128
GATHER_SPLITS = ((0, CH_TOK),)


@jax.jit
def _emb_lookup(tokens_flat, table):
    B = tokens_flat.shape[0]
    tok_per_w = B // NW
    n_ch = tok_per_w // CH_TOK

    mesh = plsc.VectorSubcoreMesh(core_axis_name="c", subcore_axis_name="s")

    @functools.partial(
        pl.kernel,
        out_type=jax.ShapeDtypeStruct((B, D), jnp.float32),
        mesh=mesh,
        compiler_params=pltpu.CompilerParams(use_tc_tiling_on_sc=True),
        scratch_types=[
            pltpu.VMEM((tok_per_w,), jnp.int32),
        ]
        + [pltpu.VMEM((CH_TOK, D), jnp.float32) for _ in range(NBUF)]
        + [pltpu.SemaphoreType.DMA for _ in range(2 * NBUF)],
    )
    def emb_kernel(tok_hbm, table_hbm, out_hbm, idx_v, *bufs_sems):
        bufs = bufs_sems[:NBUF]
        gsems = bufs_sems[NBUF:2 * NBUF]
        wsems = bufs_sems[2 * NBUF:]

        wid = lax.axis_index("s") * NC + lax.axis_index("c")
        base_tok = wid * tok_per_w
        pltpu.sync_copy(tok_hbm.at[pl.ds(base_tok, tok_per_w)], idx_v)

        def gather_descs(c, b):
            off = c * CH_TOK
            descs = [
                pltpu.make_async_copy(
                    table_hbm.at[idx_v.at[pl.ds(off + o, n)]],
                    bufs[b].at[pl.ds(o, n), :], gsems[b])
                for o, n in GATHER_SPLITS
            ]
            return descs

        def issue_gather(c, b):
            for d in gather_descs(c, b):
                d.start()

        def wait_gather(c, b):
            for d in gather_descs(c, b):
                d.wait()

        def scale(b):
            buf = bufs[b]

            def body(i, carry):
                for sub in range(D // LANES):
                    sl = pl.ds(sub * LANES, LANES)
                    buf[i, sl] = buf[i, sl] * SCALE
                return carry

            lax.fori_loop(0, CH_TOK, body, 0, unroll=2)

        def write_desc(c, b):
            return pltpu.make_async_copy(
                bufs[b],
                out_hbm.at[pl.ds(base_tok + c * CH_TOK, CH_TOK)], wsems[b])

        def step(c, b):
            wait_gather(c, b)
            scale(b)
            write_desc(c, b).start()

        for c in range(LA):
            issue_gather(c, c % NBUF)

        for c in range(NBUF):
            step(c, c)
            cn = c + LA
            if cn < NBUF:
                issue_gather(cn, cn)
            else:
                bn = cn % NBUF
                write_desc(cn - NBUF, bn).wait()
                issue_gather(cn, bn)

        def group(g, carry):
            c0 = g * NBUF
            for b in range(NBUF):
                c = c0 + b
                step(c, b)
                cn = c + LA
                bn = (b + LA) % NBUF

                @pl.when(cn < n_ch)
                def _():
                    write_desc(cn - NBUF, bn).wait()
                    issue_gather(cn, bn)
            return carry

        lax.fori_loop(1, n_ch // NBUF, group, 0)

        for b in range(NBUF):
            write_desc(0, b).wait()

    return emb_kernel(tokens_flat, table)


def kernel(tokens, table):
    rows, cols = tokens.shape
    tok_t = tokens.T.reshape(rows * cols).astype(jnp.int32)
    out = _emb_lookup(tok_t, table)
    return out.reshape(cols, rows, D).transpose(1, 0, 2)


# speedup vs baseline: 1.2469x; 1.0029x over previous
import functools
import math

import jax
import jax.numpy as jnp
from jax import lax
from jax.experimental import pallas as pl
from jax.experimental.pallas import tpu as pltpu
from jax.experimental.pallas import tpu_sc as plsc

D = 128
SCALE = math.sqrt(float(D))
NC = 2
NS = 16
NW = NC * NS
LANES = 16

NBUF = 5
LA = 3
CH_TOK = 128
GATHER_SPLITS = ((0, CH_TOK),)


@jax.jit
def _emb_lookup(tokens_flat, table):
    B = tokens_flat.shape[0]
    tok_per_w = B // NW
    n_ch = tok_per_w // CH_TOK

    mesh = plsc.VectorSubcoreMesh(core_axis_name="c", subcore_axis_name="s")

    @functools.partial(
        pl.kernel,
        out_type=jax.ShapeDtypeStruct((B, D), jnp.float32),
        mesh=mesh,
        compiler_params=pltpu.CompilerParams(use_tc_tiling_on_sc=True),
        scratch_types=[
            pltpu.VMEM((tok_per_w,), jnp.int32),
        ]
        + [pltpu.VMEM((CH_TOK, D), jnp.float32) for _ in range(NBUF)]
        + [pltpu.SemaphoreType.DMA for _ in range(2 * NBUF)],
    )
    def emb_kernel(tok_hbm, table_hbm, out_hbm, idx_v, *bufs_sems):
        bufs = bufs_sems[:NBUF]
        gsems = bufs_sems[NBUF:2 * NBUF]
        wsems = bufs_sems[2 * NBUF:]

        wid = lax.axis_index("s") * NC + lax.axis_index("c")
        base_tok = wid * tok_per_w
        pltpu.sync_copy(tok_hbm.at[pl.ds(base_tok, tok_per_w)], idx_v)

        def gather_descs(c, b):
            off = c * CH_TOK
            descs = [
                pltpu.make_async_copy(
                    table_hbm.at[idx_v.at[pl.ds(off + o, n)]],
                    bufs[b].at[pl.ds(o, n), :], gsems[b])
                for o, n in GATHER_SPLITS
            ]
            return descs

        def issue_gather(c, b):
            for d in gather_descs(c, b):
                d.start()

        def wait_gather(c, b):
            for d in gather_descs(c, b):
                d.wait()

        def scale(b):
            buf = bufs[b]

            def body(i, carry):
                for sub in range(D // LANES):
                    sl = pl.ds(sub * LANES, LANES)
                    buf[i, sl] = buf[i, sl] * SCALE
                return carry

            lax.fori_loop(0, CH_TOK, body, 0, unroll=2)

        def write_desc(c, b):
            return pltpu.make_async_copy(
                bufs[b],
                out_hbm.at[pl.ds(base_tok + c * CH_TOK, CH_TOK)], wsems[b])

        def step(c, b):
            wait_gather(c, b)
            scale(b)
            write_desc(c, b).start()

        for c in range(LA):
            issue_gather(c, c % NBUF)

        for c in range(NBUF):
            step(c, c)
            cn = c + LA
            if cn < NBUF:
                issue_gather(cn, cn)
            else:
                bn = cn % NBUF
                write_desc(cn - NBUF, bn).wait()
                issue_gather(cn, bn)

        def group(g, carry):
            c0 = g * NBUF
            for b in range(NBUF):
                c = c0 + b
                step(c, b)
                cn = c + LA
                bn = (b + LA) % NBUF

                @pl.when(cn < n_ch)
                def _():
                    write_desc(cn - NBUF, bn).wait()
                    issue_gather(cn, bn)
            return carry

        lax.fori_loop(1, n_ch // NBUF, group, 0)

        for b in range(NBUF):
            write_desc(0, b).wait()

    return emb_kernel(tokens_flat, table)


def kernel(tokens, table):
    rows, cols = tokens.shape
    tok_t = tokens.T.reshape(rows * cols).astype(jnp.int32)
    out = _emb_lookup(tok_t, table)
    return out.reshape(cols, rows, D).transpose(1, 0, 2)
